# baseline (speedup 1.0000x reference)
import functools

import jax
import jax.numpy as jnp
from jax import lax
from jax.experimental import pallas as pl
from jax.experimental.pallas import tpu as pltpu
from jax.experimental.pallas import tpu_sc as plsc

NUM_CORES = 2
NUM_SUBCORES = 16
LANES = 16
NW = NUM_CORES * NUM_SUBCORES

CHUNK = 128


def _offset_kernel(table, idx3):
    batch = idx3.shape[0] // 3
    dim = table.shape[1]
    b_per_w = batch // NW
    chunks_per_w = b_per_w // CHUNK
    mesh = plsc.VectorSubcoreMesh(core_axis_name="c", subcore_axis_name="s")

    @functools.partial(
        pl.kernel,
        out_type=jax.ShapeDtypeStruct((batch, dim), jnp.float32),
        mesh=mesh,
        scratch_types=[
            pltpu.VMEM((b_per_w,), jnp.int32),
            pltpu.VMEM((b_per_w,), jnp.int32),
            pltpu.VMEM((b_per_w,), jnp.int32),
            pltpu.VMEM((CHUNK, dim), jnp.float32),
            pltpu.VMEM((CHUNK, dim), jnp.float32),
            pltpu.VMEM((CHUNK, dim), jnp.float32),
            pltpu.SemaphoreType.DMA,
        ],
        compiler_params=pltpu.CompilerParams(use_tc_tiling_on_sc=False),
    )
    def k(table_hbm, idx_hbm, out_hbm, i1_v, i2_v, i4_v, a_v, b_v, c_v, sem):
        wid = lax.axis_index("s") * NUM_CORES + lax.axis_index("c")
        base = wid * b_per_w
        ibase = wid * (3 * b_per_w)
        pltpu.sync_copy(idx_hbm.at[pl.ds(ibase, b_per_w)], i1_v)
        pltpu.sync_copy(idx_hbm.at[pl.ds(ibase + b_per_w, b_per_w)], i2_v)
        pltpu.sync_copy(idx_hbm.at[pl.ds(ibase + 2 * b_per_w, b_per_w)], i4_v)

        @pl.loop(0, chunks_per_w)
        def _(g):
            off = g * CHUNK
            ca = pltpu.async_copy(
                table_hbm.at[i1_v.at[pl.ds(off, CHUNK)]], a_v, sem)
            cb = pltpu.async_copy(
                table_hbm.at[i2_v.at[pl.ds(off, CHUNK)]], b_v, sem)
            cc = pltpu.async_copy(
                table_hbm.at[i4_v.at[pl.ds(off, CHUNK)]], c_v, sem)
            ca.wait()
            cb.wait()
            cc.wait()

            @pl.loop(0, CHUNK)
            def _(r):
                for c in range(0, dim, LANES):
                    sl = pl.ds(c, LANES)
                    a_v[r, sl] = (a_v[r, sl] - b_v[r, sl]) + c_v[r, sl]

            pltpu.sync_copy(a_v, out_hbm.at[pl.ds(base + off, CHUNK)])

    return k(table, idx3)


def kernel(inputs, table):
    e1 = inputs[:, 0]
    e2 = inputs[:, 1]
    e3 = inputs[:, 2]
    e4 = inputs[:, 3]
    idx3 = jnp.stack([e1, e2, e4], axis=0)
    idx3 = idx3.reshape(3, NW, -1).transpose(1, 0, 2).reshape(-1)
    offset_trick = _offset_kernel(table, idx3)
    filters = jnp.concatenate(
        [e1.reshape(-1, 1), e2.reshape(-1, 1), e4.reshape(-1, 1)], axis=1)
    return (e1, e2, e3, e4, offset_trick, filters)
